# baseline (speedup 1.0000x reference)
import functools

import jax
import jax.numpy as jnp
from jax import lax
from jax.experimental import pallas as pl
from jax.experimental.pallas import tpu as pltpu
from jax.experimental.pallas import tpu_sc as plsc

D_MODEL = 64
SCALE = 8.0
C = 128


def kernel(x, table):
    out_shape = (*x.shape, D_MODEL)
    B = x.size

    info = plsc.get_sparse_core_info()
    NC, NS = info.num_cores, info.num_subcores
    NW = NC * NS
    BPW = B // NW
    NCH = BPW // C
    assert BPW * NW == B and NCH * C == BPW and NCH % 2 == 0

    x_rows = jnp.reshape(x.astype(jnp.int32), (NW * NCH, C))

    mesh = plsc.VectorSubcoreMesh(core_axis_name="c", subcore_axis_name="s")

    @functools.partial(
        pl.kernel,
        mesh=mesh,
        out_type=jax.ShapeDtypeStruct((B, D_MODEL), jnp.float32),
        compiler_params=pltpu.CompilerParams(use_tc_tiling_on_sc=False),
        scratch_types=[
            pltpu.VMEM((NCH, C), jnp.int32),
            pltpu.VMEM((2, C, D_MODEL), jnp.float32),
            pltpu.SemaphoreType.DMA,
            pltpu.SemaphoreType.DMA,
        ],
    )
    def emb(x_hbm, table_hbm, out_hbm, idx_all, rows, sem0, sem1):
        wid = lax.axis_index("c") * NS + lax.axis_index("s")
        pltpu.sync_copy(x_hbm.at[pl.ds(wid * NCH, NCH)], idx_all)

        sems = (sem0, sem1)

        def gather_start(n, b):
            pltpu.make_async_copy(
                table_hbm.at[idx_all.at[n]], rows.at[b], sems[b]
            ).start()

        def gather_wait(n, b):
            pltpu.make_async_copy(
                table_hbm.at[idx_all.at[n]], rows.at[b], sems[b]
            ).wait()

        gather_start(0, 0)

        out_base = wid * BPW

        def outer(i, _):
            n0 = i * 2
            for b in range(2):
                n = n0 + b
                nxt = n + 1

                @pl.when(nxt < NCH)
                def _():
                    gather_start(nxt, 1 - b)

                gather_wait(n, b)

                rb = rows.at[b]

                def scale4(r4, _):
                    r = r4 * 4
                    for rr in range(4):
                        for j in range(4):
                            sl = pl.ds(j * 16, 16)
                            rb[r + rr, sl] = rb[r + rr, sl] * SCALE
                    return 0

                lax.fori_loop(0, C // 4, scale4, 0)

                pltpu.sync_copy(rb, out_hbm.at[pl.ds(out_base + n * C, C)])
            return 0

        lax.fori_loop(0, NCH // 2, outer, 0)

    out = emb(x_rows, table)
    return out.reshape(out_shape)
